# baseline (speedup 1.0000x reference)
import jax
import jax.numpy as jnp
from jax import lax
from jax.experimental import pallas as pl
from jax.experimental.pallas import tpu as pltpu
from jax.experimental.pallas import tpu_sc as plsc

T, A, N, D, C = 4, 50000, 100000, 64, 64
TA = T * A
TN = T * N
S = 14336
NPASS = 14
NSLC = 16
SLC = TA // NSLC
PW = 12512
NG = PW // 16
NB = 99
PADIDX = 1 << 20
YPAD = TA + 32

_SQRT_HALF = 0.7071067811865476


def _sc_body(xf, basef, fidxp, y, idxv, vb, locb, posb, dstb, rbuf, acc):
    c = lax.axis_index("c")
    s = lax.axis_index("s")
    wid = s * 2 + c
    i16 = lax.iota(jnp.int32, 16)
    ones = i16 >= 0

    pltpu.sync_copy(fidxp.at[s], idxv)

    def one_pass(p, _):
        cb = p * (2 * S) + c * S

        def grp(g, cur):
            v = idxv[pl.ds(g * 16, 16)]
            loc = v - cb
            m = plsc.bitcast(loc, jnp.uint32) < jnp.uint32(S)
            mi = m.astype(jnp.int32)
            dest = cur + plsc.cumsum(mi) - mi
            di = dest >> 7
            dj = dest & 127
            pv = s * SLC + g * 16 + i16
            plsc.store_scatter(vb, [di, dj], v, mask=m)
            plsc.store_scatter(locb, [di, dj], loc, mask=m)
            plsc.store_scatter(posb, [di, dj], pv, mask=m)
            plsc.store_scatter(dstb, [di, dj], pv, mask=m)
            return cur + jnp.sum(mi)

        cur = lax.fori_loop(0, NG, grp, jnp.int32(0))

        def padq(q, _):
            dest = cur + q * 16 + i16
            di = dest >> 7
            dj = dest & 127
            plsc.store_scatter(vb, [di, dj],
                               jnp.zeros((16,), jnp.int32), mask=ones)
            plsc.store_scatter(locb, [di, dj],
                               jnp.full((16,), S, jnp.int32) + s, mask=ones)
            plsc.store_scatter(posb, [di, dj],
                               jnp.zeros((16,), jnp.int32), mask=ones)
            plsc.store_scatter(dstb, [di, dj],
                               jnp.full((16,), TA, jnp.int32) + wid, mask=ones)
            return 0

        lax.fori_loop(0, 8, padq, 0)
        nf = (cur + 127) >> 7

        def seed(f, _):
            pltpu.sync_copy(basef.at[vb.at[f]], rbuf)
            pltpu.sync_copy(rbuf, acc.at[locb.at[f]])
            return 0

        lax.fori_loop(0, nf, seed, 0)
        plsc.subcore_barrier()

        def scat(f, _):
            pltpu.sync_copy(xf.at[posb.at[f]], rbuf)
            pltpu.sync_copy(rbuf, acc.at[locb.at[f]], add=True)
            return 0

        lax.fori_loop(0, nf, scat, 0)
        plsc.subcore_barrier()

        def gath(f, _):
            pltpu.sync_copy(acc.at[locb.at[f]], rbuf)
            pltpu.sync_copy(rbuf, y.at[dstb.at[f]])
            return 0

        lax.fori_loop(0, nf, gath, 0)
        plsc.subcore_barrier()
        return 0

    lax.fori_loop(0, NPASS, one_pass, 0)


def _scatter_gather(xf, basef, fidxp):
    mesh = plsc.VectorSubcoreMesh(core_axis_name="c", subcore_axis_name="s")
    f = pl.kernel(
        _sc_body,
        mesh=mesh,
        compiler_params=pltpu.CompilerParams(needs_layout_passes=False,
                                             use_tc_tiling_on_sc=False),
        out_type=jax.ShapeDtypeStruct((YPAD, D), jnp.float32),
        scratch_types=[
            pltpu.VMEM((PW,), jnp.int32),
            pltpu.VMEM((NB, 128), jnp.int32),
            pltpu.VMEM((NB, 128), jnp.int32),
            pltpu.VMEM((NB, 128), jnp.int32),
            pltpu.VMEM((NB, 128), jnp.int32),
            pltpu.VMEM((128, D), jnp.float32),
            pltpu.VMEM_SHARED((S + 16, D), jnp.float32),
        ],
    )
    return f(xf, basef, fidxp)


def _gelu_exact(v):
    return 0.5 * v * (1.0 + jax.lax.erf(v * _SQRT_HALF))


def _ln(v, g, b, eps=1e-5):
    mu = jnp.mean(v, axis=-1, keepdims=True)
    var = jnp.mean((v - mu) ** 2, axis=-1, keepdims=True)
    return (v - mu) * jax.lax.rsqrt(var + eps) * g + b


def _mlp_body(y_ref, gd_ref, bd_ref, W1_ref, b1_ref, W2_ref, b2_ref,
              gu_ref, bu_ref, U1_ref, c1_ref, U2_ref, c2_ref, out_ref):
    y = y_ref[...]
    h = _ln(y, gd_ref[...], bd_ref[...])
    h = _gelu_exact(jnp.dot(h, W1_ref[...], preferred_element_type=jnp.float32)
                    + b1_ref[...])
    e = jnp.dot(h, W2_ref[...], preferred_element_type=jnp.float32) + b2_ref[...]
    g = _ln(e, gu_ref[...], bu_ref[...])
    g = _gelu_exact(jnp.dot(g, U1_ref[...], preferred_element_type=jnp.float32)
                    + c1_ref[...])
    out_ref[...] = (jnp.dot(g, U2_ref[...], preferred_element_type=jnp.float32)
                    + c2_ref[...])


def _fused_mlp(y, ln_d_g, ln_d_b, W1, b1, W2, b2, ln_u_g, ln_u_b, U1, c1, U2, c2,
               interpret=False):
    BR = 2000
    grid = (TA // BR,)
    full = lambda shape: pl.BlockSpec(shape, lambda i: (0, 0))
    return pl.pallas_call(
        _mlp_body,
        grid=grid,
        in_specs=[
            pl.BlockSpec((BR, D), lambda i: (i, 0)),
            full((1, D)), full((1, D)),
            full((D, 2 * D)), full((1, 2 * D)),
            full((2 * D, C)), full((1, C)),
            full((1, C)), full((1, C)),
            full((C, 2 * C)), full((1, 2 * C)),
            full((2 * C, D)), full((1, D)),
        ],
        out_specs=pl.BlockSpec((BR, D), lambda i: (i, 0)),
        out_shape=jax.ShapeDtypeStruct((TA, D), jnp.float32),
        interpret=interpret,
    )(y, ln_d_g.reshape(1, D), ln_d_b.reshape(1, D), W1, b1.reshape(1, 2 * D),
      W2, b2.reshape(1, C), ln_u_g.reshape(1, C), ln_u_b.reshape(1, C),
      U1, c1.reshape(1, 2 * C), U2, c2.reshape(1, D))


def kernel(x, base, ln_d_g, ln_d_b, W1, b1, W2, b2, ln_u_g, ln_u_b, U1, c1, U2, c2, indices):
    idx = indices.astype(jnp.int32)
    fidx = (idx + jnp.arange(T, dtype=jnp.int32)[:, None] * N).reshape(NSLC, SLC)
    fidxp = jnp.pad(fidx, ((0, 0), (0, PW - SLC)), constant_values=PADIDX)
    xf = x.reshape(TA, D)
    basef = base.reshape(TN, D)
    y = _scatter_gather(xf, basef, fidxp)
    return _fused_mlp(y, ln_d_g, ln_d_b, W1, b1, W2, b2,
                      ln_u_g, ln_u_b, U1, c1, U2, c2)


# speedup vs baseline: 1.1245x; 1.1245x over previous
import jax
import jax.numpy as jnp
from jax import lax
from jax.experimental import pallas as pl
from jax.experimental.pallas import tpu as pltpu
from jax.experimental.pallas import tpu_sc as plsc

T, A, N, D, C = 4, 50000, 100000, 64, 64
TA = T * A
TN = T * N
S = 12800
NPASS = 16
R = S // 16
NSLC = 16
SLC = TA // NSLC
PW = 12512
NG = PW // 16
NB = 100
PADIDX = 1 << 20
YPAD = TA + 32

_SQRT_HALF = 0.7071067811865476


def _sc_body(xf, basef, fidxp, y, idxv, locb, posb, dstb, rbuf0, rbuf1, acc,
             isem, g0, g1, s0, s1):
    c = lax.axis_index("c")
    s = lax.axis_index("s")
    wid = s * 2 + c
    i16 = lax.iota(jnp.int32, 16)
    ones = i16 >= 0

    pltpu.sync_copy(fidxp.at[s], idxv)

    def one_pass(p, _):
        cb = p * (2 * S) + c * S
        start = cb + s * R

        @pl.when(start < TN)
        def _():
            pltpu.async_copy(basef.at[pl.ds(start, R)],
                             acc.at[pl.ds(s * R, R)], isem)

        def grp(g, cur):
            v = idxv[pl.ds(g * 16, 16)]
            loc = v - cb
            m = plsc.bitcast(loc, jnp.uint32) < jnp.uint32(S)
            mi = m.astype(jnp.int32)
            dest = cur + plsc.cumsum(mi) - mi
            di = dest >> 7
            dj = dest & 127
            pv = s * SLC + g * 16 + i16
            plsc.store_scatter(locb, [di, dj], loc, mask=m)
            plsc.store_scatter(posb, [di, dj], pv, mask=m)
            plsc.store_scatter(dstb, [di, dj], pv, mask=m)
            return cur + jnp.sum(mi)

        cur = lax.fori_loop(0, NG, grp, jnp.int32(0))

        def padq(q, _):
            dest = cur + q * 16 + i16
            di = dest >> 7
            dj = dest & 127
            plsc.store_scatter(locb, [di, dj],
                               jnp.full((16,), S, jnp.int32) + s, mask=ones)
            plsc.store_scatter(posb, [di, dj],
                               jnp.zeros((16,), jnp.int32), mask=ones)
            plsc.store_scatter(dstb, [di, dj],
                               jnp.full((16,), TA, jnp.int32) + wid, mask=ones)
            return 0

        lax.fori_loop(0, 16, padq, 0)
        nf2 = (cur + 255) >> 8

        @pl.when(start < TN)
        def _():
            pltpu.make_async_copy(basef.at[pl.ds(start, R)],
                                  acc.at[pl.ds(s * R, R)], isem).wait()
        plsc.subcore_barrier()

        def scat(f2, _):
            f0 = f2 * 2
            h0 = pltpu.async_copy(xf.at[posb.at[f0]], rbuf0, g0)
            h1 = pltpu.async_copy(xf.at[posb.at[f0 + 1]], rbuf1, g1)
            h0.wait()
            a0 = pltpu.async_copy(rbuf0, acc.at[locb.at[f0]], s0, add=True)
            h1.wait()
            a1 = pltpu.async_copy(rbuf1, acc.at[locb.at[f0 + 1]], s1, add=True)
            a0.wait()
            a1.wait()
            return 0

        lax.fori_loop(0, nf2, scat, 0)
        plsc.subcore_barrier()

        def gath(f2, _):
            f0 = f2 * 2
            h0 = pltpu.async_copy(acc.at[locb.at[f0]], rbuf0, g0)
            h1 = pltpu.async_copy(acc.at[locb.at[f0 + 1]], rbuf1, g1)
            h0.wait()
            a0 = pltpu.async_copy(rbuf0, y.at[dstb.at[f0]], s0)
            h1.wait()
            a1 = pltpu.async_copy(rbuf1, y.at[dstb.at[f0 + 1]], s1)
            a0.wait()
            a1.wait()
            return 0

        lax.fori_loop(0, nf2, gath, 0)
        plsc.subcore_barrier()
        return 0

    lax.fori_loop(0, NPASS, one_pass, 0)


def _scatter_gather(xf, basef, fidxp):
    mesh = plsc.VectorSubcoreMesh(core_axis_name="c", subcore_axis_name="s")
    f = pl.kernel(
        _sc_body,
        mesh=mesh,
        compiler_params=pltpu.CompilerParams(needs_layout_passes=False,
                                             use_tc_tiling_on_sc=False),
        out_type=jax.ShapeDtypeStruct((YPAD, D), jnp.float32),
        scratch_types=[
            pltpu.VMEM((PW,), jnp.int32),
            pltpu.VMEM((NB, 128), jnp.int32),
            pltpu.VMEM((NB, 128), jnp.int32),
            pltpu.VMEM((NB, 128), jnp.int32),
            pltpu.VMEM((128, D), jnp.float32),
            pltpu.VMEM((128, D), jnp.float32),
            pltpu.VMEM_SHARED((S + 16, D), jnp.float32),
            pltpu.SemaphoreType.DMA,
            pltpu.SemaphoreType.DMA,
            pltpu.SemaphoreType.DMA,
            pltpu.SemaphoreType.DMA,
            pltpu.SemaphoreType.DMA,
        ],
    )
    return f(xf, basef, fidxp)


def _gelu_exact(v):
    return 0.5 * v * (1.0 + jax.lax.erf(v * _SQRT_HALF))


def _ln(v, g, b, eps=1e-5):
    mu = jnp.mean(v, axis=-1, keepdims=True)
    var = jnp.mean((v - mu) ** 2, axis=-1, keepdims=True)
    return (v - mu) * jax.lax.rsqrt(var + eps) * g + b


def _mlp_body(y_ref, gd_ref, bd_ref, W1_ref, b1_ref, W2_ref, b2_ref,
              gu_ref, bu_ref, U1_ref, c1_ref, U2_ref, c2_ref, out_ref):
    y = y_ref[...]
    h = _ln(y, gd_ref[...], bd_ref[...])
    h = _gelu_exact(jnp.dot(h, W1_ref[...], preferred_element_type=jnp.float32)
                    + b1_ref[...])
    e = jnp.dot(h, W2_ref[...], preferred_element_type=jnp.float32) + b2_ref[...]
    g = _ln(e, gu_ref[...], bu_ref[...])
    g = _gelu_exact(jnp.dot(g, U1_ref[...], preferred_element_type=jnp.float32)
                    + c1_ref[...])
    out_ref[...] = (jnp.dot(g, U2_ref[...], preferred_element_type=jnp.float32)
                    + c2_ref[...])


def _fused_mlp(y, ln_d_g, ln_d_b, W1, b1, W2, b2, ln_u_g, ln_u_b, U1, c1, U2, c2,
               interpret=False):
    BR = 2000
    grid = (TA // BR,)
    full = lambda shape: pl.BlockSpec(shape, lambda i: (0, 0))
    return pl.pallas_call(
        _mlp_body,
        grid=grid,
        in_specs=[
            pl.BlockSpec((BR, D), lambda i: (i, 0)),
            full((1, D)), full((1, D)),
            full((D, 2 * D)), full((1, 2 * D)),
            full((2 * D, C)), full((1, C)),
            full((1, C)), full((1, C)),
            full((C, 2 * C)), full((1, 2 * C)),
            full((2 * C, D)), full((1, D)),
        ],
        out_specs=pl.BlockSpec((BR, D), lambda i: (i, 0)),
        out_shape=jax.ShapeDtypeStruct((TA, D), jnp.float32),
        interpret=interpret,
    )(y, ln_d_g.reshape(1, D), ln_d_b.reshape(1, D), W1, b1.reshape(1, 2 * D),
      W2, b2.reshape(1, C), ln_u_g.reshape(1, C), ln_u_b.reshape(1, C),
      U1, c1.reshape(1, 2 * C), U2, c2.reshape(1, D))


def kernel(x, base, ln_d_g, ln_d_b, W1, b1, W2, b2, ln_u_g, ln_u_b, U1, c1, U2, c2, indices):
    idx = indices.astype(jnp.int32)
    fidx = (idx + jnp.arange(T, dtype=jnp.int32)[:, None] * N).reshape(NSLC, SLC)
    fidxp = jnp.pad(fidx, ((0, 0), (0, PW - SLC)), constant_values=PADIDX)
    xf = x.reshape(TA, D)
    basef = base.reshape(TN, D)
    y = _scatter_gather(xf, basef, fidxp)
    return _fused_mlp(y, ln_d_g, ln_d_b, W1, b1, W2, b2,
                      ln_u_g, ln_u_b, U1, c1, U2, c2)
